# Initial kernel scaffold; baseline (speedup 1.0000x reference)
import jax
import jax.numpy as jnp
from jax.experimental import pallas as pl


def kernel(x, edge_index, W0, aS0, aD0, b0, W1, aS1, aD1, b1, W2, aS2, aD2, b2, centers):
    raise NotImplementedError("write your pallas kernel here")


# speedup vs baseline: 1.0244x; 1.0244x over previous
import functools

import jax
import jax.numpy as jnp
from jax.experimental import pallas as pl

_N = 10000
_HEADS = 4
_HID = 256
_EMB = 16


def _proj_body(x_ref, w_ref, asm_ref, adm_ref, xp_ref, als_ref, ald_ref):
    xp = jnp.dot(x_ref[...], w_ref[...], preferred_element_type=jnp.float32)
    xp_ref[...] = xp
    als_ref[...] = jnp.dot(xp, asm_ref[...], preferred_element_type=jnp.float32)
    ald_ref[...] = jnp.dot(xp, adm_ref[...], preferred_element_type=jnp.float32)


def _project(x, W, aS, aD, bn=1000):
    n, d_in = x.shape
    d_out = W.shape[1]
    H = aS.shape[0]
    F = aS.shape[1]
    ASm = (aS[:, :, None] * jnp.eye(H, dtype=jnp.float32)[:, None, :]).reshape(H * F, H)
    ADm = (aD[:, :, None] * jnp.eye(H, dtype=jnp.float32)[:, None, :]).reshape(H * F, H)
    grid = n // bn
    return pl.pallas_call(
        _proj_body,
        grid=(grid,),
        in_specs=[
            pl.BlockSpec((bn, d_in), lambda i: (i, 0)),
            pl.BlockSpec((d_in, d_out), lambda i: (0, 0)),
            pl.BlockSpec((d_out, H), lambda i: (0, 0)),
            pl.BlockSpec((d_out, H), lambda i: (0, 0)),
        ],
        out_specs=[
            pl.BlockSpec((bn, d_out), lambda i: (i, 0)),
            pl.BlockSpec((bn, H), lambda i: (i, 0)),
            pl.BlockSpec((bn, H), lambda i: (i, 0)),
        ],
        out_shape=[
            jax.ShapeDtypeStruct((n, d_out), jnp.float32),
            jax.ShapeDtypeStruct((n, H), jnp.float32),
            jax.ShapeDtypeStruct((n, H), jnp.float32),
        ],
    )(x, W, ASm, ADm)


def _edge_aggregate(xp, als, ald, src, dst, H, F):
    e = jax.nn.leaky_relu(als[src] + ald[dst], negative_slope=0.2)
    m = jax.ops.segment_max(e, dst, num_segments=_N)
    ee = jnp.exp(e - m[dst])
    s = jax.ops.segment_sum(ee, dst, num_segments=_N)
    alpha = ee / (s[dst] + 1e-16)
    xph = xp.reshape(-1, H, F)
    out = jax.ops.segment_sum(xph[src] * alpha[:, :, None], dst, num_segments=_N)
    return out.reshape(-1, H * F)


def _dec_body(z_ref, zt_ref, ct_ref, adj_ref, q_ref):
    zb = z_ref[...]
    logits = jnp.dot(zb, zt_ref[...], preferred_element_type=jnp.float32)
    adj_ref[...] = jax.nn.sigmoid(logits)
    ct = ct_ref[...]
    zn = jnp.sum(zb * zb, axis=1, keepdims=True)
    cn = jnp.sum(ct * ct, axis=0)[None, :]
    d2 = zn + cn - 2.0 * jnp.dot(zb, ct, preferred_element_type=jnp.float32)
    q = 1.0 / (1.0 + d2)
    q_ref[...] = q / jnp.sum(q, axis=1, keepdims=True)


def _decode(z, centers, bn=400):
    n, emb = z.shape
    ncl = centers.shape[0]
    grid = n // bn
    return pl.pallas_call(
        _dec_body,
        grid=(grid,),
        in_specs=[
            pl.BlockSpec((bn, emb), lambda i: (i, 0)),
            pl.BlockSpec((emb, n), lambda i: (0, 0)),
            pl.BlockSpec((emb, ncl), lambda i: (0, 0)),
        ],
        out_specs=[
            pl.BlockSpec((bn, n), lambda i: (i, 0)),
            pl.BlockSpec((bn, ncl), lambda i: (i, 0)),
        ],
        out_shape=[
            jax.ShapeDtypeStruct((n, n), jnp.float32),
            jax.ShapeDtypeStruct((n, ncl), jnp.float32),
        ],
    )(z, z.T, centers.T)


def kernel(x, edge_index, W0, aS0, aD0, b0, W1, aS1, aD1, b1, W2, aS2, aD2, b2, centers):
    loop = jnp.arange(_N, dtype=edge_index.dtype)
    src = jnp.concatenate([edge_index[0], loop])
    dst = jnp.concatenate([edge_index[1], loop])

    xp, als, ald = _project(x, W0, aS0, aD0)
    h = jax.nn.elu(_edge_aggregate(xp, als, ald, src, dst, _HEADS, _HID) + b0)

    xp, als, ald = _project(h, W1, aS1, aD1)
    h = jax.nn.elu(_edge_aggregate(xp, als, ald, src, dst, _HEADS, _HID) + b1)

    xp, als, ald = _project(h, W2, aS2, aD2)
    z = _edge_aggregate(xp, als, ald, src, dst, 1, _EMB) + b2

    adj_recon, q = _decode(z, centers)
    return (z, adj_recon, q)


# speedup vs baseline: 1.1739x; 1.1459x over previous
import functools

import jax
import jax.numpy as jnp
from jax import lax
from jax.experimental import pallas as pl
from jax.experimental.pallas import tpu as pltpu
from jax.experimental.pallas import tpu_sc as plsc

_N = 10000
_E = 320000
_ETOT = _N + _E
_HEADS = 4
_HID = 256
_EMB = 16

_NC, _NS = 2, 16
_NW = _NC * _NS
_BPAD = ((_ETOT + 8 * _NW - 1) // (8 * _NW)) * (8 * _NW)


def _sc_gather(table, idx, win):
    V, D = table.shape
    B = idx.shape[0]
    b_per_w = B // _NW
    nwin = b_per_w // win
    assert b_per_w % win == 0 and nwin % 2 == 0 and win % 8 == 0 and win <= 128
    mesh = plsc.VectorSubcoreMesh(core_axis_name="c", subcore_axis_name="s")

    @functools.partial(
        pl.kernel,
        out_type=jax.ShapeDtypeStruct((B, D), jnp.float32),
        mesh=mesh,
        scratch_types=[
            pltpu.VMEM((2, win), jnp.int32),
            pltpu.VMEM((2, win, D), jnp.float32),
            pltpu.SemaphoreType.DMA,
            pltpu.SemaphoreType.DMA,
        ],
    )
    def gk(table_hbm, idx_hbm, out_hbm, idx_v, rows_v, sem0, sem1):
        wid = lax.axis_index("s") * _NC + lax.axis_index("c")
        base = wid * b_per_w
        sems = (sem0, sem1)

        for b in range(2):
            pltpu.sync_copy(idx_hbm.at[pl.ds(base + b * win, win)], idx_v.at[b])
            pltpu.async_copy(table_hbm.at[idx_v.at[b]], rows_v.at[b], sems[b])

        def body(w2, carry):
            for b in range(2):
                w = w2 * 2 + b
                pltpu.make_async_copy(
                    table_hbm.at[idx_v.at[b]], rows_v.at[b], sems[b]
                ).wait()
                pltpu.sync_copy(rows_v.at[b], out_hbm.at[pl.ds(base + w * win, win)])

                @pl.when(w + 2 < nwin)
                def _():
                    off = base + (w + 2) * win
                    pltpu.sync_copy(idx_hbm.at[pl.ds(off, win)], idx_v.at[b])
                    pltpu.async_copy(table_hbm.at[idx_v.at[b]], rows_v.at[b], sems[b])

            return carry

        lax.fori_loop(0, nwin // 2, body, 0)

    return gk(table, idx)


def _proj_body(x_ref, w_ref, asm_ref, adm_ref, xp_ref, als_ref, ald_ref):
    xp = jnp.dot(x_ref[...], w_ref[...], preferred_element_type=jnp.float32)
    xp_ref[...] = xp
    als_ref[...] = jnp.dot(xp, asm_ref[...], preferred_element_type=jnp.float32)
    ald_ref[...] = jnp.dot(xp, adm_ref[...], preferred_element_type=jnp.float32)


def _project(x, W, aS, aD, bn=1000):
    n, d_in = x.shape
    d_out = W.shape[1]
    H = aS.shape[0]
    F = aS.shape[1]
    ASm = (aS[:, :, None] * jnp.eye(H, dtype=jnp.float32)[:, None, :]).reshape(H * F, H)
    ADm = (aD[:, :, None] * jnp.eye(H, dtype=jnp.float32)[:, None, :]).reshape(H * F, H)
    grid = n // bn
    return pl.pallas_call(
        _proj_body,
        grid=(grid,),
        in_specs=[
            pl.BlockSpec((bn, d_in), lambda i: (i, 0)),
            pl.BlockSpec((d_in, d_out), lambda i: (0, 0)),
            pl.BlockSpec((d_out, H), lambda i: (0, 0)),
            pl.BlockSpec((d_out, H), lambda i: (0, 0)),
        ],
        out_specs=[
            pl.BlockSpec((bn, d_out), lambda i: (i, 0)),
            pl.BlockSpec((bn, H), lambda i: (i, 0)),
            pl.BlockSpec((bn, H), lambda i: (i, 0)),
        ],
        out_shape=[
            jax.ShapeDtypeStruct((n, d_out), jnp.float32),
            jax.ShapeDtypeStruct((n, H), jnp.float32),
            jax.ShapeDtypeStruct((n, H), jnp.float32),
        ],
    )(x, W, ASm, ADm)


def _edge_aggregate(xp, als, ald, srcp, dstp, dst, H, F):
    H128 = ((H + 127) // 128) * 128
    als_p = jnp.pad(als, ((0, 0), (0, H128 - H)))
    ald_p = jnp.pad(ald, ((0, 0), (0, H128 - H)))
    gs = _sc_gather(als_p, srcp, 120)[:_ETOT, :H]
    gd = _sc_gather(ald_p, dstp, 120)[:_ETOT, :H]
    e = jax.nn.leaky_relu(gs + gd, negative_slope=0.2)
    bound = jax.nn.leaky_relu(jnp.max(als) + jnp.max(ald), negative_slope=0.2)
    ee = jnp.exp(e - bound)
    s = jax.ops.segment_sum(ee, dst, num_segments=_N)
    D = H * F
    if D % 128:
        D128 = ((D + 127) // 128) * 128
        xp_g = jnp.pad(xp, ((0, 0), (0, D128 - D)))
    else:
        xp_g = xp
    big_win = 40 if xp_g.shape[1] >= 1024 else 120
    rows = _sc_gather(xp_g, srcp, big_win)[:_ETOT, :D].reshape(-1, H, F)
    out = jax.ops.segment_sum(rows * ee[:, :, None], dst, num_segments=_N)
    out = out / (s[:, :, None] + 1e-16)
    return out.reshape(-1, H * F)


def _dec_body(z_ref, zt_ref, ct_ref, adj_ref, q_ref):
    zb = z_ref[...]
    logits = jnp.dot(zb, zt_ref[...], preferred_element_type=jnp.float32)
    adj_ref[...] = jax.nn.sigmoid(logits)
    ct = ct_ref[...]
    zn = jnp.sum(zb * zb, axis=1, keepdims=True)
    cn = jnp.sum(ct * ct, axis=0)[None, :]
    d2 = zn + cn - 2.0 * jnp.dot(zb, ct, preferred_element_type=jnp.float32)
    q = 1.0 / (1.0 + d2)
    q_ref[...] = q / jnp.sum(q, axis=1, keepdims=True)


def _decode(z, centers, bn=400):
    n, emb = z.shape
    ncl = centers.shape[0]
    grid = n // bn
    return pl.pallas_call(
        _dec_body,
        grid=(grid,),
        in_specs=[
            pl.BlockSpec((bn, emb), lambda i: (i, 0)),
            pl.BlockSpec((emb, n), lambda i: (0, 0)),
            pl.BlockSpec((emb, ncl), lambda i: (0, 0)),
        ],
        out_specs=[
            pl.BlockSpec((bn, n), lambda i: (i, 0)),
            pl.BlockSpec((bn, ncl), lambda i: (i, 0)),
        ],
        out_shape=[
            jax.ShapeDtypeStruct((n, n), jnp.float32),
            jax.ShapeDtypeStruct((n, ncl), jnp.float32),
        ],
    )(z, z.T, centers.T)


def kernel(x, edge_index, W0, aS0, aD0, b0, W1, aS1, aD1, b1, W2, aS2, aD2, b2, centers):
    loop = jnp.arange(_N, dtype=edge_index.dtype)
    src = jnp.concatenate([edge_index[0], loop])
    dst = jnp.concatenate([edge_index[1], loop])
    pad = jnp.arange(_BPAD - _ETOT, dtype=edge_index.dtype)
    srcp = jnp.concatenate([src, pad])
    dstp = jnp.concatenate([dst, pad])

    xp, als, ald = _project(x, W0, aS0, aD0)
    h = jax.nn.elu(_edge_aggregate(xp, als, ald, srcp, dstp, dst, _HEADS, _HID) + b0)

    xp, als, ald = _project(h, W1, aS1, aD1)
    h = jax.nn.elu(_edge_aggregate(xp, als, ald, srcp, dstp, dst, _HEADS, _HID) + b1)

    xp, als, ald = _project(h, W2, aS2, aD2)
    z = _edge_aggregate(xp, als, ald, srcp, dstp, dst, 1, _EMB) + b2

    adj_recon, q = _decode(z, centers)
    return (z, adj_recon, q)


# speedup vs baseline: 7.5897x; 6.4655x over previous
import functools

import jax
import jax.numpy as jnp
from jax import lax
from jax.experimental import pallas as pl
from jax.experimental.pallas import tpu as pltpu
from jax.experimental.pallas import tpu_sc as plsc

_N = 10000
_E = 320000
_ETOT = _N + _E
_HEADS = 4
_HID = 256
_EMB = 16

_NC, _NS = 2, 16
_NW = _NC * _NS
_BPAD = ((_ETOT + 8 * _NW - 1) // (8 * _NW)) * (8 * _NW)


def _sc_gather(table, idx, win):
    V, D = table.shape
    B = idx.shape[0]
    b_per_w = B // _NW
    nwin = b_per_w // win
    assert b_per_w % win == 0 and nwin % 2 == 0 and win % 8 == 0 and win <= 128
    mesh = plsc.VectorSubcoreMesh(core_axis_name="c", subcore_axis_name="s")

    @functools.partial(
        pl.kernel,
        out_type=jax.ShapeDtypeStruct((B, D), jnp.float32),
        mesh=mesh,
        scratch_types=[
            pltpu.VMEM((2, win), jnp.int32),
            pltpu.VMEM((2, win, D), jnp.float32),
            pltpu.SemaphoreType.DMA,
            pltpu.SemaphoreType.DMA,
        ],
    )
    def gk(table_hbm, idx_hbm, out_hbm, idx_v, rows_v, sem0, sem1):
        wid = lax.axis_index("s") * _NC + lax.axis_index("c")
        base = wid * b_per_w
        sems = (sem0, sem1)

        for b in range(2):
            pltpu.sync_copy(idx_hbm.at[pl.ds(base + b * win, win)], idx_v.at[b])
            pltpu.async_copy(table_hbm.at[idx_v.at[b]], rows_v.at[b], sems[b])

        def body(w2, carry):
            for b in range(2):
                w = w2 * 2 + b
                pltpu.make_async_copy(
                    table_hbm.at[idx_v.at[b]], rows_v.at[b], sems[b]
                ).wait()
                pltpu.sync_copy(rows_v.at[b], out_hbm.at[pl.ds(base + w * win, win)])

                @pl.when(w + 2 < nwin)
                def _():
                    off = base + (w + 2) * win
                    pltpu.sync_copy(idx_hbm.at[pl.ds(off, win)], idx_v.at[b])
                    pltpu.async_copy(table_hbm.at[idx_v.at[b]], rows_v.at[b], sems[b])

            return carry

        lax.fori_loop(0, nwin // 2, body, 0)

    return gk(table, idx)


def _sc_aggregate_1024(xp, eeT, srcp, dstp):
    W = 48
    EPT = _BPAD // _NS
    NWIN = EPT // W
    mesh = plsc.VectorSubcoreMesh(core_axis_name="c", subcore_axis_name="s")

    xp8 = xp.reshape(_N * 8, 128)
    srcp_r = srcp.reshape(_NS, NWIN, W)
    dstp_r = dstp.reshape(_NS, NWIN, W)
    eeT_r = eeT.reshape(_HEADS, _NS, NWIN, W)
    zeros = jnp.zeros((640, 128), jnp.float32)

    @functools.partial(
        pl.kernel,
        out_type=jax.ShapeDtypeStruct((_N, 1024), jnp.float32),
        mesh=mesh,
        scratch_types=[
            pltpu.VMEM_SHARED((_N, 128), jnp.float32),
            pltpu.VMEM((2, W), jnp.int32),
            pltpu.VMEM((2, W), jnp.int32),
            pltpu.VMEM((2, W), jnp.float32),
            pltpu.VMEM((2, W), jnp.int32),
            pltpu.VMEM((2, W, 128), jnp.float32),
            pltpu.VMEM((2, W, 128), jnp.float32),
            pltpu.SemaphoreType.DMA,
            pltpu.SemaphoreType.DMA,
            pltpu.SemaphoreType.DMA,
            pltpu.SemaphoreType.DMA,
        ],
    )
    def agg(xp8_hbm, srcp_hbm, dstp_hbm, eeT_hbm, zeros_hbm, out_hbm,
            accum, idx_v, dst_v, ee_v, sdst, rows_in, rows_out,
            gsem0, gsem1, ssem0, ssem1):
        c = lax.axis_index("c")
        sid = lax.axis_index("s")
        gsems = (gsem0, gsem1)
        ssems = (ssem0, ssem1)

        def bcast(av, l):
            return lax.gather(
                av,
                jnp.full((16, 1), l, jnp.int32),
                lax.GatherDimensionNumbers(
                    offset_dims=(),
                    collapsed_slice_dims=(0,),
                    start_index_map=(0,),
                ),
                (1,),
                mode=lax.GatherScatterMode.PROMISE_IN_BOUNDS,
            )

        for jj in range(4):
            jg = c * 4 + jj
            h = 2 * c + (jj // 2)

            pltpu.sync_copy(zeros_hbm, accum.at[pl.ds(sid * 624, 640)])
            plsc.subcore_barrier()

            def load_window(b, w):
                pltpu.sync_copy(srcp_hbm.at[sid, w], idx_v.at[b])
                pltpu.sync_copy(dstp_hbm.at[sid, w], dst_v.at[b])
                pltpu.sync_copy(eeT_hbm.at[h, sid, w], ee_v.at[b])
                for g in range(W // 16):
                    v = idx_v[b, pl.ds(g * 16, 16)]
                    idx_v[b, pl.ds(g * 16, 16)] = v * 8 + jg

            for b in range(2):
                load_window(b, b)
                pltpu.async_copy(
                    xp8_hbm.at[idx_v.at[b]], rows_in.at[b], gsems[b]
                )

            def window(w2, carry):
                for b in range(2):
                    w = w2 * 2 + b
                    pltpu.make_async_copy(
                        xp8_hbm.at[idx_v.at[b]], rows_in.at[b], gsems[b]
                    ).wait()

                    @pl.when(w >= 2)
                    def _():
                        pltpu.make_async_copy(
                            rows_out.at[b], accum.at[sdst.at[b]], ssems[b]
                        ).wait()

                    def mgrp(g, carry2):
                        av = ee_v[b, pl.ds(g * 16, 16)]
                        for l in range(16):
                            a = bcast(av, l)
                            r = g * 16 + l
                            for cv in range(8):
                                rows_out[b, r, pl.ds(cv * 16, 16)] = (
                                    rows_in[b, r, pl.ds(cv * 16, 16)] * a
                                )
                        return carry2

                    lax.fori_loop(0, W // 16, mgrp, 0)
                    for g in range(W // 16):
                        sdst[b, pl.ds(g * 16, 16)] = dst_v[b, pl.ds(g * 16, 16)]
                    pltpu.async_copy(
                        rows_out.at[b], accum.at[sdst.at[b]], ssems[b],
                        add=True,
                    )

                    @pl.when(w + 2 < NWIN)
                    def _():
                        load_window(b, w + 2)
                        pltpu.async_copy(
                            xp8_hbm.at[idx_v.at[b]], rows_in.at[b], gsems[b]
                        )

                return carry

            lax.fori_loop(0, NWIN // 2, window, 0)
            for b in range(2):
                pltpu.make_async_copy(
                    rows_out.at[b], accum.at[sdst.at[b]], ssems[b]
                ).wait()
            plsc.subcore_barrier()

            pltpu.sync_copy(
                accum.at[pl.ds(sid * 624, 624)],
                out_hbm.at[pl.ds(sid * 624, 624), pl.ds(jg * 128, 128)],
            )

            @pl.when(sid == _NS - 1)
            def _():
                pltpu.sync_copy(
                    accum.at[pl.ds(9984, 16)],
                    out_hbm.at[pl.ds(9984, 16), pl.ds(jg * 128, 128)],
                )

            plsc.subcore_barrier()

    return agg(xp8, srcp_r, dstp_r, eeT_r, zeros)


def _proj_body(x_ref, w_ref, asm_ref, adm_ref, xp_ref, als_ref, ald_ref):
    xp = jnp.dot(x_ref[...], w_ref[...], preferred_element_type=jnp.float32)
    xp_ref[...] = xp
    als_ref[...] = jnp.dot(xp, asm_ref[...], preferred_element_type=jnp.float32)
    ald_ref[...] = jnp.dot(xp, adm_ref[...], preferred_element_type=jnp.float32)


def _project(x, W, aS, aD, bn=1000):
    n, d_in = x.shape
    d_out = W.shape[1]
    H = aS.shape[0]
    F = aS.shape[1]
    ASm = (aS[:, :, None] * jnp.eye(H, dtype=jnp.float32)[:, None, :]).reshape(H * F, H)
    ADm = (aD[:, :, None] * jnp.eye(H, dtype=jnp.float32)[:, None, :]).reshape(H * F, H)
    grid = n // bn
    return pl.pallas_call(
        _proj_body,
        grid=(grid,),
        in_specs=[
            pl.BlockSpec((bn, d_in), lambda i: (i, 0)),
            pl.BlockSpec((d_in, d_out), lambda i: (0, 0)),
            pl.BlockSpec((d_out, H), lambda i: (0, 0)),
            pl.BlockSpec((d_out, H), lambda i: (0, 0)),
        ],
        out_specs=[
            pl.BlockSpec((bn, d_out), lambda i: (i, 0)),
            pl.BlockSpec((bn, H), lambda i: (i, 0)),
            pl.BlockSpec((bn, H), lambda i: (i, 0)),
        ],
        out_shape=[
            jax.ShapeDtypeStruct((n, d_out), jnp.float32),
            jax.ShapeDtypeStruct((n, H), jnp.float32),
            jax.ShapeDtypeStruct((n, H), jnp.float32),
        ],
    )(x, W, ASm, ADm)


def _edge_aggregate(xp, als, ald, srcp, dstp, dst, H, F):
    H128 = ((H + 127) // 128) * 128
    als_p = jnp.pad(als, ((0, 0), (0, H128 - H)))
    ald_p = jnp.pad(ald, ((0, 0), (0, H128 - H)))
    gs = _sc_gather(als_p, srcp, 120)[:_ETOT, :H]
    gd = _sc_gather(ald_p, dstp, 120)[:_ETOT, :H]
    e = jax.nn.leaky_relu(gs + gd, negative_slope=0.2)
    bound = jax.nn.leaky_relu(jnp.max(als) + jnp.max(ald), negative_slope=0.2)
    ee = jnp.exp(e - bound)
    s = jax.ops.segment_sum(ee, dst, num_segments=_N)
    D = H * F
    if D == 1024:
        eeT = jnp.pad(ee, ((0, _BPAD - _ETOT), (0, 0))).T
        out = _sc_aggregate_1024(xp, eeT, srcp, dstp).reshape(-1, H, F)
    else:
        D128 = ((D + 127) // 128) * 128
        xp_g = jnp.pad(xp, ((0, 0), (0, D128 - D))) if D % 128 else xp
        rows = _sc_gather(xp_g, srcp, 120)[:_ETOT, :D].reshape(-1, H, F)
        out = jax.ops.segment_sum(rows * ee[:, :, None], dst, num_segments=_N)
    out = out / (s[:, :, None] + 1e-16)
    return out.reshape(-1, H * F)


def _dec_body(z_ref, zt_ref, ct_ref, adj_ref, q_ref):
    zb = z_ref[...]
    logits = jnp.dot(zb, zt_ref[...], preferred_element_type=jnp.float32)
    adj_ref[...] = jax.nn.sigmoid(logits)
    ct = ct_ref[...]
    zn = jnp.sum(zb * zb, axis=1, keepdims=True)
    cn = jnp.sum(ct * ct, axis=0)[None, :]
    d2 = zn + cn - 2.0 * jnp.dot(zb, ct, preferred_element_type=jnp.float32)
    q = 1.0 / (1.0 + d2)
    q_ref[...] = q / jnp.sum(q, axis=1, keepdims=True)


def _decode(z, centers, bn=400):
    n, emb = z.shape
    ncl = centers.shape[0]
    grid = n // bn
    return pl.pallas_call(
        _dec_body,
        grid=(grid,),
        in_specs=[
            pl.BlockSpec((bn, emb), lambda i: (i, 0)),
            pl.BlockSpec((emb, n), lambda i: (0, 0)),
            pl.BlockSpec((emb, ncl), lambda i: (0, 0)),
        ],
        out_specs=[
            pl.BlockSpec((bn, n), lambda i: (i, 0)),
            pl.BlockSpec((bn, ncl), lambda i: (i, 0)),
        ],
        out_shape=[
            jax.ShapeDtypeStruct((n, n), jnp.float32),
            jax.ShapeDtypeStruct((n, ncl), jnp.float32),
        ],
    )(z, z.T, centers.T)


def kernel(x, edge_index, W0, aS0, aD0, b0, W1, aS1, aD1, b1, W2, aS2, aD2, b2, centers):
    loop = jnp.arange(_N, dtype=edge_index.dtype)
    src = jnp.concatenate([edge_index[0], loop])
    dst = jnp.concatenate([edge_index[1], loop])
    pad = jnp.arange(_BPAD - _ETOT, dtype=edge_index.dtype)
    srcp = jnp.concatenate([src, pad])
    dstp = jnp.concatenate([dst, pad])

    xp, als, ald = _project(x, W0, aS0, aD0)
    h = jax.nn.elu(_edge_aggregate(xp, als, ald, srcp, dstp, dst, _HEADS, _HID) + b0)

    xp, als, ald = _project(h, W1, aS1, aD1)
    h = jax.nn.elu(_edge_aggregate(xp, als, ald, srcp, dstp, dst, _HEADS, _HID) + b1)

    xp, als, ald = _project(h, W2, aS2, aD2)
    z = _edge_aggregate(xp, als, ald, srcp, dstp, dst, 1, _EMB) + b2

    adj_recon, q = _decode(z, centers)
    return (z, adj_recon, q)


# speedup vs baseline: 8.2295x; 1.0843x over previous
import functools

import jax
import jax.numpy as jnp
from jax import lax
from jax.experimental import pallas as pl
from jax.experimental.pallas import tpu as pltpu
from jax.experimental.pallas import tpu_sc as plsc

_N = 10000
_E = 320000
_ETOT = _N + _E
_HEADS = 4
_HID = 256
_EMB = 16

_NC, _NS = 2, 16
_NW = _NC * _NS
_BPAD = ((_ETOT + 8 * _NW - 1) // (8 * _NW)) * (8 * _NW)
_BPAD2 = 331776


def _sc_gather(table, idx, win):
    V, D = table.shape
    B = idx.shape[0]
    b_per_w = B // _NW
    nwin = b_per_w // win
    assert b_per_w % win == 0 and nwin % 2 == 0 and win % 8 == 0 and win <= 128
    mesh = plsc.VectorSubcoreMesh(core_axis_name="c", subcore_axis_name="s")

    @functools.partial(
        pl.kernel,
        out_type=jax.ShapeDtypeStruct((B, D), jnp.float32),
        mesh=mesh,
        scratch_types=[
            pltpu.VMEM((2, win), jnp.int32),
            pltpu.VMEM((2, win, D), jnp.float32),
            pltpu.SemaphoreType.DMA,
            pltpu.SemaphoreType.DMA,
        ],
    )
    def gk(table_hbm, idx_hbm, out_hbm, idx_v, rows_v, sem0, sem1):
        wid = lax.axis_index("s") * _NC + lax.axis_index("c")
        base = wid * b_per_w
        sems = (sem0, sem1)

        for b in range(2):
            pltpu.sync_copy(idx_hbm.at[pl.ds(base + b * win, win)], idx_v.at[b])
            pltpu.async_copy(table_hbm.at[idx_v.at[b]], rows_v.at[b], sems[b])

        def body(w2, carry):
            for b in range(2):
                w = w2 * 2 + b
                pltpu.make_async_copy(
                    table_hbm.at[idx_v.at[b]], rows_v.at[b], sems[b]
                ).wait()
                pltpu.sync_copy(rows_v.at[b], out_hbm.at[pl.ds(base + w * win, win)])

                @pl.when(w + 2 < nwin)
                def _():
                    off = base + (w + 2) * win
                    pltpu.sync_copy(idx_hbm.at[pl.ds(off, win)], idx_v.at[b])
                    pltpu.async_copy(table_hbm.at[idx_v.at[b]], rows_v.at[b], sems[b])

            return carry

        lax.fori_loop(0, nwin // 2, body, 0)

    return gk(table, idx)


def _sc_aggregate_1024(xp, eeT, srcp, dstp):
    W = 64
    EPT = _BPAD2 // _NS
    NWIN = EPT // W
    mesh = plsc.VectorSubcoreMesh(core_axis_name="c", subcore_axis_name="s")

    xp8 = xp.reshape(_N * 8, 128)
    srcp_r = srcp.reshape(_NS, NWIN, W)
    dstp_r = dstp.reshape(_NS, NWIN, W)
    eeT_r = eeT.reshape(_HEADS, _NS, NWIN, W)
    zeros = jnp.zeros((640, 128), jnp.float32)

    @functools.partial(
        pl.kernel,
        out_type=jax.ShapeDtypeStruct((_N, 1024), jnp.float32),
        mesh=mesh,
        scratch_types=[
            pltpu.VMEM_SHARED((_N, 128), jnp.float32),
            pltpu.VMEM((2, W), jnp.int32),
            pltpu.VMEM((2, W), jnp.int32),
            pltpu.VMEM((2, W), jnp.float32),
            pltpu.VMEM((2, W), jnp.int32),
            pltpu.VMEM((2, W, 128), jnp.float32),
            pltpu.VMEM((2, W, 128), jnp.float32),
            pltpu.SemaphoreType.DMA,
            pltpu.SemaphoreType.DMA,
            pltpu.SemaphoreType.DMA,
            pltpu.SemaphoreType.DMA,
        ],
    )
    def agg(xp8_hbm, srcp_hbm, dstp_hbm, eeT_hbm, zeros_hbm, out_hbm,
            accum, idx_v, dst_v, ee_v, sdst, rows_in, rows_out,
            gsem0, gsem1, ssem0, ssem1):
        c = lax.axis_index("c")
        sid = lax.axis_index("s")
        gsems = (gsem0, gsem1)
        ssems = (ssem0, ssem1)

        def bcast(av, l):
            return lax.gather(
                av,
                jnp.full((16, 1), l, jnp.int32),
                lax.GatherDimensionNumbers(
                    offset_dims=(),
                    collapsed_slice_dims=(0,),
                    start_index_map=(0,),
                ),
                (1,),
                mode=lax.GatherScatterMode.PROMISE_IN_BOUNDS,
            )

        for jj in range(4):
            jg = c * 4 + jj
            h = 2 * c + (jj // 2)

            pltpu.sync_copy(zeros_hbm, accum.at[pl.ds(sid * 624, 640)])
            plsc.subcore_barrier()

            def load_window(b, w):
                pltpu.sync_copy(srcp_hbm.at[sid, w], idx_v.at[b])
                pltpu.sync_copy(dstp_hbm.at[sid, w], dst_v.at[b])
                pltpu.sync_copy(eeT_hbm.at[h, sid, w], ee_v.at[b])
                for g in range(W // 16):
                    v = idx_v[b, pl.ds(g * 16, 16)]
                    idx_v[b, pl.ds(g * 16, 16)] = v * 8 + jg

            for b in range(2):
                load_window(b, b)
                pltpu.async_copy(
                    xp8_hbm.at[idx_v.at[b]], rows_in.at[b], gsems[b]
                )

            def window(w2, carry):
                for b in range(2):
                    w = w2 * 2 + b
                    pltpu.make_async_copy(
                        xp8_hbm.at[idx_v.at[b]], rows_in.at[b], gsems[b]
                    ).wait()

                    @pl.when(w >= 2)
                    def _():
                        pltpu.make_async_copy(
                            rows_out.at[b], accum.at[sdst.at[b]], ssems[b]
                        ).wait()

                    def mgrp(g, carry2):
                        av = ee_v[b, pl.ds(g * 16, 16)]
                        for l in range(16):
                            a = bcast(av, l)
                            r = g * 16 + l
                            for cv in range(8):
                                rows_out[b, r, pl.ds(cv * 16, 16)] = (
                                    rows_in[b, r, pl.ds(cv * 16, 16)] * a
                                )
                        return carry2

                    lax.fori_loop(0, W // 16, mgrp, 0)
                    for g in range(W // 16):
                        sdst[b, pl.ds(g * 16, 16)] = dst_v[b, pl.ds(g * 16, 16)]
                    pltpu.async_copy(
                        rows_out.at[b], accum.at[sdst.at[b]], ssems[b],
                        add=True,
                    )

                    @pl.when(w + 2 < NWIN)
                    def _():
                        load_window(b, w + 2)
                        pltpu.async_copy(
                            xp8_hbm.at[idx_v.at[b]], rows_in.at[b], gsems[b]
                        )

                return carry

            lax.fori_loop(0, NWIN // 2, window, 0)
            for b in range(2):
                pltpu.make_async_copy(
                    rows_out.at[b], accum.at[sdst.at[b]], ssems[b]
                ).wait()
            plsc.subcore_barrier()

            pltpu.sync_copy(
                accum.at[pl.ds(sid * 624, 624)],
                out_hbm.at[pl.ds(sid * 624, 624), pl.ds(jg * 128, 128)],
            )

            @pl.when(sid == _NS - 1)
            def _():
                pltpu.sync_copy(
                    accum.at[pl.ds(9984, 16)],
                    out_hbm.at[pl.ds(9984, 16), pl.ds(jg * 128, 128)],
                )

            plsc.subcore_barrier()

    return agg(xp8, srcp_r, dstp_r, eeT_r, zeros)


def _proj_body(x_ref, w_ref, asm_ref, adm_ref, xp_ref, als_ref, ald_ref):
    xp = jnp.dot(x_ref[...], w_ref[...], preferred_element_type=jnp.float32)
    xp_ref[...] = xp
    als_ref[...] = jnp.dot(xp, asm_ref[...], preferred_element_type=jnp.float32)
    ald_ref[...] = jnp.dot(xp, adm_ref[...], preferred_element_type=jnp.float32)


def _project(x, W, aS, aD, bn=1000):
    n, d_in = x.shape
    d_out = W.shape[1]
    H = aS.shape[0]
    F = aS.shape[1]
    ASm = (aS[:, :, None] * jnp.eye(H, dtype=jnp.float32)[:, None, :]).reshape(H * F, H)
    ADm = (aD[:, :, None] * jnp.eye(H, dtype=jnp.float32)[:, None, :]).reshape(H * F, H)
    grid = n // bn
    return pl.pallas_call(
        _proj_body,
        grid=(grid,),
        in_specs=[
            pl.BlockSpec((bn, d_in), lambda i: (i, 0)),
            pl.BlockSpec((d_in, d_out), lambda i: (0, 0)),
            pl.BlockSpec((d_out, H), lambda i: (0, 0)),
            pl.BlockSpec((d_out, H), lambda i: (0, 0)),
        ],
        out_specs=[
            pl.BlockSpec((bn, d_out), lambda i: (i, 0)),
            pl.BlockSpec((bn, H), lambda i: (i, 0)),
            pl.BlockSpec((bn, H), lambda i: (i, 0)),
        ],
        out_shape=[
            jax.ShapeDtypeStruct((n, d_out), jnp.float32),
            jax.ShapeDtypeStruct((n, H), jnp.float32),
            jax.ShapeDtypeStruct((n, H), jnp.float32),
        ],
    )(x, W, ASm, ADm)


def _edge_aggregate(xp, als, ald, srcp, dstp, dst, H, F):
    H128 = ((H + 127) // 128) * 128
    als_p = jnp.pad(als, ((0, 0), (0, H128 - H)))
    ald_p = jnp.pad(ald, ((0, 0), (0, H128 - H)))
    gs = _sc_gather(als_p, srcp, 120)[:_ETOT, :H]
    gd = _sc_gather(ald_p, dstp, 120)[:_ETOT, :H]
    e = jax.nn.leaky_relu(gs + gd, negative_slope=0.2)
    bound = jax.nn.leaky_relu(jnp.max(als) + jnp.max(ald), negative_slope=0.2)
    ee = jnp.exp(e - bound)
    s = jax.ops.segment_sum(ee, dst, num_segments=_N)
    D = H * F
    if D == 1024:
        pad2 = _BPAD2 - _ETOT
        eeT = jnp.pad(ee, ((0, pad2), (0, 0))).T
        srcp2 = jnp.concatenate([srcp[:_ETOT], jnp.zeros((pad2,), srcp.dtype)])
        dstp2 = jnp.concatenate([dstp[:_ETOT], jnp.zeros((pad2,), dstp.dtype)])
        out = _sc_aggregate_1024(xp, eeT, srcp2, dstp2).reshape(-1, H, F)
    else:
        D128 = ((D + 127) // 128) * 128
        xp_g = jnp.pad(xp, ((0, 0), (0, D128 - D))) if D % 128 else xp
        rows = _sc_gather(xp_g, srcp, 120)[:_ETOT, :D].reshape(-1, H, F)
        out = jax.ops.segment_sum(rows * ee[:, :, None], dst, num_segments=_N)
    out = out / (s[:, :, None] + 1e-16)
    return out.reshape(-1, H * F)


def _dec_body(z_ref, zt_ref, ct_ref, adj_ref, q_ref):
    zb = z_ref[...]
    logits = jnp.dot(zb, zt_ref[...], preferred_element_type=jnp.float32)
    adj_ref[...] = jax.nn.sigmoid(logits)
    ct = ct_ref[...]
    zn = jnp.sum(zb * zb, axis=1, keepdims=True)
    cn = jnp.sum(ct * ct, axis=0)[None, :]
    d2 = zn + cn - 2.0 * jnp.dot(zb, ct, preferred_element_type=jnp.float32)
    q = 1.0 / (1.0 + d2)
    q_ref[...] = q / jnp.sum(q, axis=1, keepdims=True)


def _decode(z, centers, bn=400):
    n, emb = z.shape
    ncl = centers.shape[0]
    grid = n // bn
    return pl.pallas_call(
        _dec_body,
        grid=(grid,),
        in_specs=[
            pl.BlockSpec((bn, emb), lambda i: (i, 0)),
            pl.BlockSpec((emb, n), lambda i: (0, 0)),
            pl.BlockSpec((emb, ncl), lambda i: (0, 0)),
        ],
        out_specs=[
            pl.BlockSpec((bn, n), lambda i: (i, 0)),
            pl.BlockSpec((bn, ncl), lambda i: (i, 0)),
        ],
        out_shape=[
            jax.ShapeDtypeStruct((n, n), jnp.float32),
            jax.ShapeDtypeStruct((n, ncl), jnp.float32),
        ],
    )(z, z.T, centers.T)


def kernel(x, edge_index, W0, aS0, aD0, b0, W1, aS1, aD1, b1, W2, aS2, aD2, b2, centers):
    loop = jnp.arange(_N, dtype=edge_index.dtype)
    src = jnp.concatenate([edge_index[0], loop])
    dst = jnp.concatenate([edge_index[1], loop])
    pad = jnp.arange(_BPAD - _ETOT, dtype=edge_index.dtype)
    srcp = jnp.concatenate([src, pad])
    dstp = jnp.concatenate([dst, pad])

    xp, als, ald = _project(x, W0, aS0, aD0)
    h = jax.nn.elu(_edge_aggregate(xp, als, ald, srcp, dstp, dst, _HEADS, _HID) + b0)

    xp, als, ald = _project(h, W1, aS1, aD1)
    h = jax.nn.elu(_edge_aggregate(xp, als, ald, srcp, dstp, dst, _HEADS, _HID) + b1)

    xp, als, ald = _project(h, W2, aS2, aD2)
    z = _edge_aggregate(xp, als, ald, srcp, dstp, dst, 1, _EMB) + b2

    adj_recon, q = _decode(z, centers)
    return (z, adj_recon, q)


# speedup vs baseline: 10.3862x; 1.2621x over previous
import functools

import jax
import jax.numpy as jnp
from jax import lax
from jax.experimental import pallas as pl
from jax.experimental.pallas import tpu as pltpu
from jax.experimental.pallas import tpu_sc as plsc

_N = 10000
_E = 320000
_ETOT = _N + _E
_HEADS = 4
_HID = 256
_EMB = 16

_NC, _NS = 2, 16
_NW = _NC * _NS
_BPAD = ((_ETOT + 8 * _NW - 1) // (8 * _NW)) * (8 * _NW)
_BPAD2 = 331776


def _sc_gather(table, idx, win):
    V, D = table.shape
    B = idx.shape[0]
    b_per_w = B // _NW
    nwin = b_per_w // win
    assert b_per_w % win == 0 and nwin % 2 == 0 and win % 8 == 0 and win <= 128
    mesh = plsc.VectorSubcoreMesh(core_axis_name="c", subcore_axis_name="s")

    @functools.partial(
        pl.kernel,
        out_type=jax.ShapeDtypeStruct((B, D), jnp.float32),
        mesh=mesh,
        scratch_types=[
            pltpu.VMEM((2, win), jnp.int32),
            pltpu.VMEM((2, win, D), jnp.float32),
            pltpu.SemaphoreType.DMA,
            pltpu.SemaphoreType.DMA,
        ],
    )
    def gk(table_hbm, idx_hbm, out_hbm, idx_v, rows_v, sem0, sem1):
        wid = lax.axis_index("s") * _NC + lax.axis_index("c")
        base = wid * b_per_w
        sems = (sem0, sem1)

        for b in range(2):
            pltpu.sync_copy(idx_hbm.at[pl.ds(base + b * win, win)], idx_v.at[b])
            pltpu.async_copy(table_hbm.at[idx_v.at[b]], rows_v.at[b], sems[b])

        def body(w2, carry):
            for b in range(2):
                w = w2 * 2 + b
                pltpu.make_async_copy(
                    table_hbm.at[idx_v.at[b]], rows_v.at[b], sems[b]
                ).wait()
                pltpu.sync_copy(rows_v.at[b], out_hbm.at[pl.ds(base + w * win, win)])

                @pl.when(w + 2 < nwin)
                def _():
                    off = base + (w + 2) * win
                    pltpu.sync_copy(idx_hbm.at[pl.ds(off, win)], idx_v.at[b])
                    pltpu.async_copy(table_hbm.at[idx_v.at[b]], rows_v.at[b], sems[b])

            return carry

        lax.fori_loop(0, nwin // 2, body, 0)

    return gk(table, idx)


def _sc_aggregate_1024(xp, eeT, srcp, dstp):
    W = 64
    EPT = _BPAD2 // _NS
    NWIN = EPT // W
    mesh = plsc.VectorSubcoreMesh(core_axis_name="c", subcore_axis_name="s")

    xp8 = xp.reshape(_N * 8, 128)
    srcp_r = srcp.reshape(_NS, NWIN, W)
    dstp_r = dstp.reshape(_NS, NWIN, W)
    eeT_r = eeT.reshape(_HEADS, _NS, NWIN, W)
    zeros = jnp.zeros((640, 128), jnp.float32)
    zeros1 = jnp.zeros((640,), jnp.float32)

    @functools.partial(
        pl.kernel,
        out_type=(
            jax.ShapeDtypeStruct((_N, 1024), jnp.float32),
            jax.ShapeDtypeStruct((_HEADS, 1, 10240), jnp.float32),
        ),
        mesh=mesh,
        scratch_types=[
            pltpu.VMEM_SHARED((_N, 128), jnp.float32),
            pltpu.VMEM_SHARED((10240,), jnp.float32),
            pltpu.VMEM_SHARED((10240,), jnp.float32),
            pltpu.VMEM((2, W), jnp.int32),
            pltpu.VMEM((2, W), jnp.int32),
            pltpu.VMEM((2, W), jnp.float32),
            pltpu.VMEM((2, W), jnp.int32),
            pltpu.VMEM((2, W), jnp.float32),
            pltpu.VMEM((2, W, 128), jnp.float32),
            pltpu.VMEM((2, W, 128), jnp.float32),
            pltpu.SemaphoreType.DMA,
            pltpu.SemaphoreType.DMA,
            pltpu.SemaphoreType.DMA,
            pltpu.SemaphoreType.DMA,
            pltpu.SemaphoreType.DMA,
            pltpu.SemaphoreType.DMA,
        ],
    )
    def agg(xp8_hbm, srcp_hbm, dstp_hbm, eeT_hbm, zeros_hbm, zeros1_hbm,
            out_hbm, s_out_hbm,
            accum, s_acc0, s_acc1, idx_v, dst_v, ee_v, sdst, see,
            rows_in, rows_out,
            gsem0, gsem1, ssem0, ssem1, tsem0, tsem1):
        c = lax.axis_index("c")
        sid = lax.axis_index("s")
        gsems = (gsem0, gsem1)
        ssems = (ssem0, ssem1)
        tsems = (tsem0, tsem1)
        s_accs = (s_acc0, s_acc1)

        def bcast(av, l):
            return lax.gather(
                av,
                jnp.full((16, 1), l, jnp.int32),
                lax.GatherDimensionNumbers(
                    offset_dims=(),
                    collapsed_slice_dims=(0,),
                    start_index_map=(0,),
                ),
                (1,),
                mode=lax.GatherScatterMode.PROMISE_IN_BOUNDS,
            )

        for jj in range(4):
            jg = c * 4 + jj
            h = 2 * c + (jj // 2)
            se = (jj % 2 == 0)
            s_acc = s_accs[jj // 2]

            pltpu.sync_copy(zeros_hbm, accum.at[pl.ds(sid * 624, 640)])
            if se:
                pltpu.sync_copy(zeros1_hbm, s_acc.at[pl.ds(sid * 640, 640)])
            plsc.subcore_barrier()

            def load_window(b, w):
                pltpu.sync_copy(srcp_hbm.at[sid, w], idx_v.at[b])
                pltpu.sync_copy(dstp_hbm.at[sid, w], dst_v.at[b])
                pltpu.sync_copy(eeT_hbm.at[h, sid, w], ee_v.at[b])
                for g in range(W // 16):
                    v = idx_v[b, pl.ds(g * 16, 16)]
                    idx_v[b, pl.ds(g * 16, 16)] = v * 8 + jg

            for b in range(2):
                load_window(b, b)
                pltpu.async_copy(
                    xp8_hbm.at[idx_v.at[b]], rows_in.at[b], gsems[b]
                )

            def window(w2, carry):
                for b in range(2):
                    w = w2 * 2 + b
                    pltpu.make_async_copy(
                        xp8_hbm.at[idx_v.at[b]], rows_in.at[b], gsems[b]
                    ).wait()

                    @pl.when(w >= 2)
                    def _():
                        pltpu.make_async_copy(
                            rows_out.at[b], accum.at[sdst.at[b]], ssems[b]
                        ).wait()
                        if se:
                            pltpu.make_async_copy(
                                see.at[b], s_acc.at[sdst.at[b]], tsems[b]
                            ).wait()

                    def mgrp(g, carry2):
                        av = ee_v[b, pl.ds(g * 16, 16)]
                        for l in range(16):
                            a = bcast(av, l)
                            r = g * 16 + l
                            for cv in range(8):
                                rows_out[b, r, pl.ds(cv * 16, 16)] = (
                                    rows_in[b, r, pl.ds(cv * 16, 16)] * a
                                )
                        return carry2

                    lax.fori_loop(0, W // 16, mgrp, 0)
                    for g in range(W // 16):
                        sdst[b, pl.ds(g * 16, 16)] = dst_v[b, pl.ds(g * 16, 16)]
                    pltpu.async_copy(
                        rows_out.at[b], accum.at[sdst.at[b]], ssems[b],
                        add=True,
                    )
                    if se:
                        for g in range(W // 16):
                            see[b, pl.ds(g * 16, 16)] = ee_v[b, pl.ds(g * 16, 16)]
                        pltpu.async_copy(
                            see.at[b], s_acc.at[sdst.at[b]], tsems[b],
                            add=True,
                        )

                    @pl.when(w + 2 < NWIN)
                    def _():
                        load_window(b, w + 2)
                        pltpu.async_copy(
                            xp8_hbm.at[idx_v.at[b]], rows_in.at[b], gsems[b]
                        )

                return carry

            lax.fori_loop(0, NWIN // 2, window, 0)
            for b in range(2):
                pltpu.make_async_copy(
                    rows_out.at[b], accum.at[sdst.at[b]], ssems[b]
                ).wait()
                if se:
                    pltpu.make_async_copy(
                        see.at[b], s_acc.at[sdst.at[b]], tsems[b]
                    ).wait()
            plsc.subcore_barrier()

            pltpu.sync_copy(
                accum.at[pl.ds(sid * 624, 624)],
                out_hbm.at[pl.ds(sid * 624, 624), pl.ds(jg * 128, 128)],
            )

            @pl.when(sid == _NS - 1)
            def _():
                pltpu.sync_copy(
                    accum.at[pl.ds(9984, 16)],
                    out_hbm.at[pl.ds(9984, 16), pl.ds(jg * 128, 128)],
                )

            if se:
                pltpu.sync_copy(
                    s_acc.at[pl.ds(sid * 640, 640)],
                    s_out_hbm.at[h, 0, pl.ds(sid * 640, 640)],
                )

            plsc.subcore_barrier()

    return agg(xp8, srcp_r, dstp_r, eeT_r, zeros, zeros1)


def _proj_body(x_ref, w_ref, asm_ref, adm_ref, xp_ref, als_ref, ald_ref):
    xp = jnp.dot(x_ref[...], w_ref[...], preferred_element_type=jnp.float32)
    xp_ref[...] = xp
    als_ref[...] = jnp.dot(xp, asm_ref[...], preferred_element_type=jnp.float32)
    ald_ref[...] = jnp.dot(xp, adm_ref[...], preferred_element_type=jnp.float32)


def _project(x, W, aS, aD, bn=1000):
    n, d_in = x.shape
    d_out = W.shape[1]
    H = aS.shape[0]
    F = aS.shape[1]
    ASm = (aS[:, :, None] * jnp.eye(H, dtype=jnp.float32)[:, None, :]).reshape(H * F, H)
    ADm = (aD[:, :, None] * jnp.eye(H, dtype=jnp.float32)[:, None, :]).reshape(H * F, H)
    grid = n // bn
    return pl.pallas_call(
        _proj_body,
        grid=(grid,),
        in_specs=[
            pl.BlockSpec((bn, d_in), lambda i: (i, 0)),
            pl.BlockSpec((d_in, d_out), lambda i: (0, 0)),
            pl.BlockSpec((d_out, H), lambda i: (0, 0)),
            pl.BlockSpec((d_out, H), lambda i: (0, 0)),
        ],
        out_specs=[
            pl.BlockSpec((bn, d_out), lambda i: (i, 0)),
            pl.BlockSpec((bn, H), lambda i: (i, 0)),
            pl.BlockSpec((bn, H), lambda i: (i, 0)),
        ],
        out_shape=[
            jax.ShapeDtypeStruct((n, d_out), jnp.float32),
            jax.ShapeDtypeStruct((n, H), jnp.float32),
            jax.ShapeDtypeStruct((n, H), jnp.float32),
        ],
    )(x, W, ASm, ADm)


def _edge_aggregate(xp, als, ald, srcp, dstp, dst, H, F):
    H128 = ((H + 127) // 128) * 128
    als_p = jnp.pad(als, ((0, 0), (0, H128 - H)))
    ald_p = jnp.pad(ald, ((0, 0), (0, H128 - H)))
    gs = _sc_gather(als_p, srcp, 120)[:_ETOT, :H]
    gd = _sc_gather(ald_p, dstp, 120)[:_ETOT, :H]
    e = jax.nn.leaky_relu(gs + gd, negative_slope=0.2)
    bound = jax.nn.leaky_relu(jnp.max(als) + jnp.max(ald), negative_slope=0.2)
    ee = jnp.exp(e - bound)
    D = H * F
    if D == 1024:
        pad2 = _BPAD2 - _ETOT
        eeT = jnp.pad(ee, ((0, pad2), (0, 0))).T
        srcp2 = jnp.concatenate([srcp[:_ETOT], jnp.zeros((pad2,), srcp.dtype)])
        dstp2 = jnp.concatenate([dstp[:_ETOT], jnp.zeros((pad2,), dstp.dtype)])
        out2, s_out = _sc_aggregate_1024(xp, eeT, srcp2, dstp2)
        s = s_out[:, 0, :_N].T
        out = out2.reshape(-1, H, F)
    else:
        s = jax.ops.segment_sum(ee, dst, num_segments=_N)
        D128 = ((D + 127) // 128) * 128
        xp_g = jnp.pad(xp, ((0, 0), (0, D128 - D))) if D % 128 else xp
        rows = _sc_gather(xp_g, srcp, 120)[:_ETOT, :D].reshape(-1, H, F)
        out = jax.ops.segment_sum(rows * ee[:, :, None], dst, num_segments=_N)
    out = out / (s[:, :, None] + 1e-16)
    return out.reshape(-1, H * F)


def _dec_body(z_ref, zt_ref, ct_ref, adj_ref, q_ref):
    zb = z_ref[...]
    logits = jnp.dot(zb, zt_ref[...], preferred_element_type=jnp.float32)
    adj_ref[...] = jax.nn.sigmoid(logits)
    ct = ct_ref[...]
    zn = jnp.sum(zb * zb, axis=1, keepdims=True)
    cn = jnp.sum(ct * ct, axis=0)[None, :]
    d2 = zn + cn - 2.0 * jnp.dot(zb, ct, preferred_element_type=jnp.float32)
    q = 1.0 / (1.0 + d2)
    q_ref[...] = q / jnp.sum(q, axis=1, keepdims=True)


def _decode(z, centers, bn=400):
    n, emb = z.shape
    ncl = centers.shape[0]
    grid = n // bn
    return pl.pallas_call(
        _dec_body,
        grid=(grid,),
        in_specs=[
            pl.BlockSpec((bn, emb), lambda i: (i, 0)),
            pl.BlockSpec((emb, n), lambda i: (0, 0)),
            pl.BlockSpec((emb, ncl), lambda i: (0, 0)),
        ],
        out_specs=[
            pl.BlockSpec((bn, n), lambda i: (i, 0)),
            pl.BlockSpec((bn, ncl), lambda i: (i, 0)),
        ],
        out_shape=[
            jax.ShapeDtypeStruct((n, n), jnp.float32),
            jax.ShapeDtypeStruct((n, ncl), jnp.float32),
        ],
    )(z, z.T, centers.T)


def kernel(x, edge_index, W0, aS0, aD0, b0, W1, aS1, aD1, b1, W2, aS2, aD2, b2, centers):
    loop = jnp.arange(_N, dtype=edge_index.dtype)
    src = jnp.concatenate([edge_index[0], loop])
    dst = jnp.concatenate([edge_index[1], loop])
    pad = jnp.arange(_BPAD - _ETOT, dtype=edge_index.dtype)
    srcp = jnp.concatenate([src, pad])
    dstp = jnp.concatenate([dst, pad])

    xp, als, ald = _project(x, W0, aS0, aD0)
    h = jax.nn.elu(_edge_aggregate(xp, als, ald, srcp, dstp, dst, _HEADS, _HID) + b0)

    xp, als, ald = _project(h, W1, aS1, aD1)
    h = jax.nn.elu(_edge_aggregate(xp, als, ald, srcp, dstp, dst, _HEADS, _HID) + b1)

    xp, als, ald = _project(h, W2, aS2, aD2)
    z = _edge_aggregate(xp, als, ald, srcp, dstp, dst, 1, _EMB) + b2

    adj_recon, q = _decode(z, centers)
    return (z, adj_recon, q)
